# Initial kernel scaffold; baseline (speedup 1.0000x reference)
import jax
import jax.numpy as jnp
from jax.experimental import pallas as pl


def kernel(x, edge_index, W_proj, b_proj, W_l, b_l, W_r, W_fc, b_fc):
    raise NotImplementedError("write your pallas kernel here")


# speedup vs baseline: 6.0722x; 6.0722x over previous
import functools

import jax
import jax.numpy as jnp
from jax import lax
from jax.experimental import pallas as pl
from jax.experimental.pallas import tpu as pltpu
from jax.experimental.pallas import tpu_sc as plsc

NC = 2
NS = 16
LN = 16
NW = NC * NS
CHUNK = 128
CW = CHUNK // 2


def _pack_body(e_ref, out_ref):
    half = e_ref.shape[1] // 2
    a = e_ref[:, :half]
    b = e_ref[:, half:]
    out_ref[...] = a | (b << 16)


def _proj_body(x_ref, w_ref, b_ref, out_ref):
    out_ref[...] = jnp.maximum(
        jnp.dot(x_ref[...], w_ref[...], preferred_element_type=jnp.float32)
        + b_ref[...], 0.0)


def _tail_body(agg_ref, cnt_ref, x_ref, wl_ref, bl_ref, wr_ref, wfc_ref,
               bfc_ref, out_ref, *, n):
    agg = agg_ref[0, :n, :] + agg_ref[1, :n, :]
    cntv = cnt_ref[0, :n, :] + cnt_ref[1, :n, :]
    cnt = jnp.sum(cntv, axis=1, keepdims=True) * (1.0 / cntv.shape[1])
    mean = agg / jnp.maximum(cnt, 1.0)
    x = x_ref[...]
    out = (jnp.dot(mean, wl_ref[...], preferred_element_type=jnp.float32)
           + bl_ref[...]
           + jnp.dot(x, wr_ref[...], preferred_element_type=jnp.float32))
    nrm = jnp.sqrt(jnp.sum(out * out, axis=1, keepdims=True))
    out = out / jnp.maximum(nrm, 1e-12)
    h = jnp.maximum(out, 0.0)
    out_ref[...] = (jnp.dot(h, wfc_ref[...], preferred_element_type=jnp.float32)
                    + bfc_ref[...])


def _unpack_chunk(word_v, idx_v, shift_mask=True):
    for t in range(CW // LN):
        v = word_v[pl.ds(t * LN, LN)]
        idx_v[pl.ds(2 * t * LN, LN)] = v & 0xFFFF
        idx_v[pl.ds((2 * t + 1) * LN, LN)] = jnp.right_shift(v, 16)


def _make_sc_aggregate(n_sp, d, nchunk, rpt):
    mesh = plsc.VectorSubcoreMesh(core_axis_name="c", subcore_axis_name="s",
                                  num_cores=NC, num_subcores=NS)

    @functools.partial(
        pl.kernel,
        out_type=jax.ShapeDtypeStruct((NC, n_sp, d), jnp.float32),
        mesh=mesh,
        scratch_types=[
            pltpu.VMEM((CW,), jnp.int32),
            pltpu.VMEM((CW,), jnp.int32),
            pltpu.VMEM((CHUNK,), jnp.int32),
            pltpu.VMEM((CHUNK,), jnp.int32),
            pltpu.VMEM((CHUNK, d), jnp.float32),
            pltpu.VMEM_SHARED((n_sp, d), jnp.float32),
            pltpu.SemaphoreType.DMA,
        ],
    )
    def sc_aggregate(eidx_hbm, xp_hbm, agg_out,
                     srcw_v, dstw_v, srci_v, dsti_v, gbuf, agg_sh, sem):
        c = lax.axis_index("c")
        s = lax.axis_index("s")
        w = c * NS + s
        base = s * rpt
        kw = (nchunk - w + NW - 1) // NW

        def fill(i, _):
            def inner(t, __):
                gbuf[i, pl.ds(t * LN, LN)] = jnp.zeros((LN,), jnp.float32)
                return 0
            lax.fori_loop(0, d // LN, inner, 0)
            return 0
        lax.fori_loop(0, CHUNK, fill, 0)

        for off in range(0, rpt, CHUNK):
            sz = min(CHUNK, rpt - off)
            pltpu.sync_copy(gbuf.at[pl.ds(0, sz)],
                            agg_sh.at[pl.ds(base + off, sz)])
        plsc.subcore_barrier()

        def step(j, _):
            chunk = j * NW + w
            pltpu.sync_copy(eidx_hbm.at[0, pl.ds(chunk * CW, CW)], srcw_v)
            pltpu.sync_copy(eidx_hbm.at[1, pl.ds(chunk * CW, CW)], dstw_v)
            _unpack_chunk(srcw_v, srci_v)
            _unpack_chunk(dstw_v, dsti_v)
            pltpu.async_copy(xp_hbm.at[srci_v], gbuf, sem).wait()
            pltpu.sync_copy(gbuf, agg_sh.at[dsti_v], add=True)
            return 0
        lax.fori_loop(0, kw, step, 0)

        plsc.subcore_barrier()

        for off in range(0, rpt, CHUNK):
            sz = min(CHUNK, rpt - off)
            pltpu.sync_copy(agg_sh.at[pl.ds(base + off, sz)],
                            agg_out.at[c, pl.ds(base + off, sz)])

    return sc_aggregate


def _make_sc_count(n_sp, dc, nchunk, rpt):
    mesh = plsc.VectorSubcoreMesh(core_axis_name="c", subcore_axis_name="s",
                                  num_cores=NC, num_subcores=NS)

    @functools.partial(
        pl.kernel,
        out_type=jax.ShapeDtypeStruct((NC, n_sp, dc), jnp.float32),
        mesh=mesh,
        scratch_types=[
            pltpu.VMEM((CW,), jnp.int32),
            pltpu.VMEM((CHUNK,), jnp.int32),
            pltpu.VMEM((CHUNK, dc), jnp.float32),
            pltpu.VMEM((CHUNK, dc), jnp.float32),
            pltpu.VMEM_SHARED((n_sp, dc), jnp.float32),
        ],
    )
    def sc_count(eidx_hbm, dep_hbm, cnt_out, dstw_v, dsti_v, ones_v, zb,
                 cnt_sh):
        del dep_hbm
        c = lax.axis_index("c")
        s = lax.axis_index("s")
        w = c * NS + s
        base = s * rpt
        kw = (nchunk - w + NW - 1) // NW

        def fill(i, _):
            def inner(t, __):
                ones_v[i, pl.ds(t * LN, LN)] = jnp.ones((LN,), jnp.float32)
                zb[i, pl.ds(t * LN, LN)] = jnp.zeros((LN,), jnp.float32)
                return 0
            lax.fori_loop(0, dc // LN, inner, 0)
            return 0
        lax.fori_loop(0, CHUNK, fill, 0)

        for off in range(0, rpt, CHUNK):
            sz = min(CHUNK, rpt - off)
            pltpu.sync_copy(zb.at[pl.ds(0, sz)],
                            cnt_sh.at[pl.ds(base + off, sz)])
        plsc.subcore_barrier()

        def step(j, _):
            chunk = j * NW + w
            pltpu.sync_copy(eidx_hbm.at[1, pl.ds(chunk * CW, CW)], dstw_v)
            _unpack_chunk(dstw_v, dsti_v)
            pltpu.sync_copy(ones_v, cnt_sh.at[dsti_v], add=True)
            return 0
        lax.fori_loop(0, kw, step, 0)

        plsc.subcore_barrier()

        for off in range(0, rpt, CHUNK):
            sz = min(CHUNK, rpt - off)
            pltpu.sync_copy(cnt_sh.at[pl.ds(base + off, sz)],
                            cnt_out.at[c, pl.ds(base + off, sz)])

    return sc_count


def kernel(x, edge_index, W_proj, b_proj, W_l, b_l, W_r, W_fc, b_fc):
    n, d = x.shape
    e = edge_index.shape[1]
    c_out = W_fc.shape[1]

    nchunk = e // CHUNK
    assert nchunk * CHUNK == e, "edge count must be a multiple of 128"
    rpt = (-(-n // NS) + 7) // 8 * 8
    n_sp = NS * rpt

    eidx_pk = pl.pallas_call(
        _pack_body,
        out_shape=jax.ShapeDtypeStruct((2, e // 2), jnp.int32),
    )(edge_index)

    xp = pl.pallas_call(
        _proj_body,
        out_shape=jax.ShapeDtypeStruct((n, d), jnp.float32),
    )(x, W_proj, b_proj.reshape(1, d))

    agg2 = _make_sc_aggregate(n_sp, d, nchunk, rpt)(eidx_pk, xp)
    cnt2 = _make_sc_count(n_sp, d, nchunk, rpt)(eidx_pk, agg2)

    logits = pl.pallas_call(
        functools.partial(_tail_body, n=n),
        out_shape=jax.ShapeDtypeStruct((n, c_out), jnp.float32),
    )(agg2, cnt2, x, W_l, b_l.reshape(1, -1), W_r, W_fc,
      b_fc.reshape(1, -1))
    return logits


# speedup vs baseline: 8.1468x; 1.3417x over previous
import functools

import jax
import jax.numpy as jnp
from jax import lax
from jax.experimental import pallas as pl
from jax.experimental.pallas import tpu as pltpu
from jax.experimental.pallas import tpu_sc as plsc

NC = 2
NS = 16
LN = 16
NW = NC * NS
CHUNK = 128
CW = CHUNK // 2


def _pack_body(e_ref, out_ref):
    half = e_ref.shape[1] // 2
    a = e_ref[:, :half]
    b = e_ref[:, half:]
    out_ref[...] = a | (b << 16)


def _proj_body(x_ref, w_ref, b_ref, out_ref):
    out_ref[...] = jnp.maximum(
        jnp.dot(x_ref[...], w_ref[...], preferred_element_type=jnp.float32)
        + b_ref[...], 0.0)


def _tail_body(agg_ref, cnt_ref, x_ref, wl_ref, bl_ref, wr_ref, wfc_ref,
               bfc_ref, out_ref, *, n):
    agg = agg_ref[0, :n, :] + agg_ref[1, :n, :]
    cntv = cnt_ref[0, :n, :] + cnt_ref[1, :n, :]
    cnt = jnp.sum(cntv, axis=1, keepdims=True) * (1.0 / cntv.shape[1])
    mean = agg / jnp.maximum(cnt, 1.0)
    x = x_ref[...]
    out = (jnp.dot(mean, wl_ref[...], preferred_element_type=jnp.float32)
           + bl_ref[...]
           + jnp.dot(x, wr_ref[...], preferred_element_type=jnp.float32))
    nrm = jnp.sqrt(jnp.sum(out * out, axis=1, keepdims=True))
    out = out / jnp.maximum(nrm, 1e-12)
    h = jnp.maximum(out, 0.0)
    out_ref[...] = (jnp.dot(h, wfc_ref[...], preferred_element_type=jnp.float32)
                    + bfc_ref[...])


def _unpack_chunk(word_v, idx_v, shift_mask=True):
    for t in range(CW // LN):
        v = word_v[pl.ds(t * LN, LN)]
        idx_v[pl.ds(2 * t * LN, LN)] = v & 0xFFFF
        idx_v[pl.ds((2 * t + 1) * LN, LN)] = jnp.right_shift(v, 16)


def _make_sc_aggregate(n_sp, d, nchunk, rpt):
    mesh = plsc.VectorSubcoreMesh(core_axis_name="c", subcore_axis_name="s",
                                  num_cores=NC, num_subcores=NS)

    @functools.partial(
        pl.kernel,
        out_type=jax.ShapeDtypeStruct((NC, n_sp, d), jnp.float32),
        mesh=mesh,
        scratch_types=[
            pltpu.VMEM((2, CW), jnp.int32),
            pltpu.VMEM((2, CW), jnp.int32),
            pltpu.VMEM((2, CHUNK), jnp.int32),
            pltpu.VMEM((2, CHUNK), jnp.int32),
            pltpu.VMEM((2, CHUNK, d), jnp.float32),
            pltpu.VMEM_SHARED((n_sp, d), jnp.float32),
            pltpu.SemaphoreType.DMA((2,)),
        ],
    )
    def sc_aggregate(eidx_hbm, xp_hbm, agg_out,
                     srcw_v, dstw_v, srci_v, dsti_v, gbuf, agg_sh, sem):
        c = lax.axis_index("c")
        s = lax.axis_index("s")
        w = c * NS + s
        base = s * rpt
        kw = (nchunk - w + NW - 1) // NW

        def fill(i, _):
            def inner(t, __):
                gbuf[0, i, pl.ds(t * LN, LN)] = jnp.zeros((LN,), jnp.float32)
                return 0
            lax.fori_loop(0, d // LN, inner, 0)
            return 0
        lax.fori_loop(0, CHUNK, fill, 0)

        for off in range(0, rpt, CHUNK):
            sz = min(CHUNK, rpt - off)
            pltpu.sync_copy(gbuf.at[0, pl.ds(0, sz)],
                            agg_sh.at[pl.ds(base + off, sz)])
        plsc.subcore_barrier()

        def load_idx(j, b):
            chunk = j * NW + w
            pltpu.sync_copy(eidx_hbm.at[0, pl.ds(chunk * CW, CW)],
                            srcw_v.at[b])
            pltpu.sync_copy(eidx_hbm.at[1, pl.ds(chunk * CW, CW)],
                            dstw_v.at[b])
            for t in range(CW // LN):
                sv = srcw_v[b, pl.ds(t * LN, LN)]
                dv = dstw_v[b, pl.ds(t * LN, LN)]
                srci_v[b, pl.ds(2 * t * LN, LN)] = sv & 0xFFFF
                srci_v[b, pl.ds((2 * t + 1) * LN, LN)] = (
                    jnp.right_shift(sv, 16))
                dsti_v[b, pl.ds(2 * t * LN, LN)] = dv & 0xFFFF
                dsti_v[b, pl.ds((2 * t + 1) * LN, LN)] = (
                    jnp.right_shift(dv, 16))

        def start_gather(b):
            pltpu.async_copy(xp_hbm.at[srci_v.at[b]], gbuf.at[b], sem.at[b])

        def finish_scatter(b):
            pltpu.make_async_copy(xp_hbm.at[srci_v.at[b]], gbuf.at[b],
                                  sem.at[b]).wait()
            pltpu.sync_copy(gbuf.at[b], agg_sh.at[dsti_v.at[b]], add=True)

        @pl.when(kw > 0)
        def _prologue():
            load_idx(0, 0)
            start_gather(0)

        def steps(jj, _):
            j0 = 2 * jj
            j1 = 2 * jj + 1

            @pl.when(j1 < kw)
            def _():
                load_idx(j1, 1)
                start_gather(1)
            finish_scatter(0)

            @pl.when(j0 + 2 < kw)
            def _():
                load_idx(j0 + 2, 0)
                start_gather(0)

            @pl.when(j1 < kw)
            def _():
                finish_scatter(1)
            return 0
        lax.fori_loop(0, (kw + 1) // 2, steps, 0)

        plsc.subcore_barrier()

        for off in range(0, rpt, CHUNK):
            sz = min(CHUNK, rpt - off)
            pltpu.sync_copy(agg_sh.at[pl.ds(base + off, sz)],
                            agg_out.at[c, pl.ds(base + off, sz)])

    return sc_aggregate


def _make_sc_count(n_sp, dc, nchunk, rpt):
    mesh = plsc.VectorSubcoreMesh(core_axis_name="c", subcore_axis_name="s",
                                  num_cores=NC, num_subcores=NS)

    @functools.partial(
        pl.kernel,
        out_type=jax.ShapeDtypeStruct((NC, n_sp, dc), jnp.float32),
        mesh=mesh,
        scratch_types=[
            pltpu.VMEM((CW,), jnp.int32),
            pltpu.VMEM((CHUNK,), jnp.int32),
            pltpu.VMEM((CHUNK, dc), jnp.float32),
            pltpu.VMEM((CHUNK, dc), jnp.float32),
            pltpu.VMEM_SHARED((n_sp, dc), jnp.float32),
        ],
    )
    def sc_count(eidx_hbm, dep_hbm, cnt_out, dstw_v, dsti_v, ones_v, zb,
                 cnt_sh):
        del dep_hbm
        c = lax.axis_index("c")
        s = lax.axis_index("s")
        w = c * NS + s
        base = s * rpt
        kw = (nchunk - w + NW - 1) // NW

        def fill(i, _):
            def inner(t, __):
                ones_v[i, pl.ds(t * LN, LN)] = jnp.ones((LN,), jnp.float32)
                zb[i, pl.ds(t * LN, LN)] = jnp.zeros((LN,), jnp.float32)
                return 0
            lax.fori_loop(0, dc // LN, inner, 0)
            return 0
        lax.fori_loop(0, CHUNK, fill, 0)

        for off in range(0, rpt, CHUNK):
            sz = min(CHUNK, rpt - off)
            pltpu.sync_copy(zb.at[pl.ds(0, sz)],
                            cnt_sh.at[pl.ds(base + off, sz)])
        plsc.subcore_barrier()

        def step(j, _):
            chunk = j * NW + w
            pltpu.sync_copy(eidx_hbm.at[1, pl.ds(chunk * CW, CW)], dstw_v)
            _unpack_chunk(dstw_v, dsti_v)
            pltpu.sync_copy(ones_v, cnt_sh.at[dsti_v], add=True)
            return 0
        lax.fori_loop(0, kw, step, 0)

        plsc.subcore_barrier()

        for off in range(0, rpt, CHUNK):
            sz = min(CHUNK, rpt - off)
            pltpu.sync_copy(cnt_sh.at[pl.ds(base + off, sz)],
                            cnt_out.at[c, pl.ds(base + off, sz)])

    return sc_count


def kernel(x, edge_index, W_proj, b_proj, W_l, b_l, W_r, W_fc, b_fc):
    n, d = x.shape
    e = edge_index.shape[1]
    c_out = W_fc.shape[1]

    nchunk = e // CHUNK
    assert nchunk * CHUNK == e, "edge count must be a multiple of 128"
    rpt = (-(-n // NS) + 7) // 8 * 8
    n_sp = NS * rpt

    eidx_pk = pl.pallas_call(
        _pack_body,
        out_shape=jax.ShapeDtypeStruct((2, e // 2), jnp.int32),
    )(edge_index)

    xp = pl.pallas_call(
        _proj_body,
        out_shape=jax.ShapeDtypeStruct((n, d), jnp.float32),
    )(x, W_proj, b_proj.reshape(1, d))

    agg2 = _make_sc_aggregate(n_sp, d, nchunk, rpt)(eidx_pk, xp)
    cnt2 = _make_sc_count(n_sp, d, nchunk, rpt)(eidx_pk, agg2)

    logits = pl.pallas_call(
        functools.partial(_tail_body, n=n),
        out_shape=jax.ShapeDtypeStruct((n, c_out), jnp.float32),
    )(agg2, cnt2, x, W_l, b_l.reshape(1, -1), W_r, W_fc,
      b_fc.reshape(1, -1))
    return logits


# speedup vs baseline: 8.2920x; 1.0178x over previous
import functools

import jax
import jax.numpy as jnp
from jax import lax
from jax.experimental import pallas as pl
from jax.experimental.pallas import tpu as pltpu
from jax.experimental.pallas import tpu_sc as plsc

NC = 2
NS = 16
LN = 16
NW = NC * NS
CHUNK = 128
CW = CHUNK // 2


def _pack_body(e_ref, out_ref):
    half = e_ref.shape[1] // 2
    a = e_ref[:, :half]
    b = e_ref[:, half:]
    out_ref[...] = a | (b << 16)


def _proj_body(x_ref, w_ref, b_ref, out_ref):
    out_ref[...] = jnp.maximum(
        jnp.dot(x_ref[...], w_ref[...], preferred_element_type=jnp.float32)
        + b_ref[...], 0.0)


def _tail_body(agg_ref, cnt_ref, x_ref, wl_ref, bl_ref, wr_ref, wfc_ref,
               bfc_ref, out_ref, *, n):
    agg = agg_ref[0, :n, :] + agg_ref[1, :n, :]
    cntv = cnt_ref[0, :n, :] + cnt_ref[1, :n, :]
    cnt = jnp.sum(cntv, axis=1, keepdims=True) * (1.0 / cntv.shape[1])
    mean = agg / jnp.maximum(cnt, 1.0)
    x = x_ref[...]
    out = (jnp.dot(mean, wl_ref[...], preferred_element_type=jnp.float32)
           + bl_ref[...]
           + jnp.dot(x, wr_ref[...], preferred_element_type=jnp.float32))
    nrm = jnp.sqrt(jnp.sum(out * out, axis=1, keepdims=True))
    out = out / jnp.maximum(nrm, 1e-12)
    h = jnp.maximum(out, 0.0)
    out_ref[...] = (jnp.dot(h, wfc_ref[...], preferred_element_type=jnp.float32)
                    + bfc_ref[...])


def _unpack_chunk(word_v, idx_v, shift_mask=True):
    for t in range(CW // LN):
        v = word_v[pl.ds(t * LN, LN)]
        idx_v[pl.ds(2 * t * LN, LN)] = v & 0xFFFF
        idx_v[pl.ds((2 * t + 1) * LN, LN)] = jnp.right_shift(v, 16)


def _make_sc_aggregate(n_sp, d, nchunk, rpt):
    mesh = plsc.VectorSubcoreMesh(core_axis_name="c", subcore_axis_name="s",
                                  num_cores=NC, num_subcores=NS)

    @functools.partial(
        pl.kernel,
        out_type=jax.ShapeDtypeStruct((NC, n_sp, d), jnp.float32),
        mesh=mesh,
        scratch_types=[
            pltpu.VMEM((2, CW), jnp.int32),
            pltpu.VMEM((2, CW), jnp.int32),
            pltpu.VMEM((2, CHUNK), jnp.int32),
            pltpu.VMEM((2, CHUNK), jnp.int32),
            pltpu.VMEM((2, CHUNK, d), jnp.float32),
            pltpu.VMEM_SHARED((n_sp, d), jnp.float32),
            pltpu.SemaphoreType.DMA((2,)),
        ],
    )
    def sc_aggregate(eidx_hbm, xp_hbm, dep_hbm, agg_out,
                     srcw_v, dstw_v, srci_v, dsti_v, gbuf, agg_sh, sem):
        del dep_hbm
        c = lax.axis_index("c")
        s = lax.axis_index("s")
        w = c * NS + s
        base = s * rpt
        kw = (nchunk - w + NW - 1) // NW

        def fill(i, _):
            def inner(t, __):
                gbuf[0, i, pl.ds(t * LN, LN)] = jnp.zeros((LN,), jnp.float32)
                return 0
            lax.fori_loop(0, d // LN, inner, 0)
            return 0
        lax.fori_loop(0, CHUNK, fill, 0)

        for off in range(0, rpt, CHUNK):
            sz = min(CHUNK, rpt - off)
            pltpu.sync_copy(gbuf.at[0, pl.ds(0, sz)],
                            agg_sh.at[pl.ds(base + off, sz)])
        plsc.subcore_barrier()

        def load_idx(j, b):
            chunk = j * NW + w
            pltpu.sync_copy(eidx_hbm.at[0, pl.ds(chunk * CW, CW)],
                            srcw_v.at[b])
            pltpu.sync_copy(eidx_hbm.at[1, pl.ds(chunk * CW, CW)],
                            dstw_v.at[b])
            for t in range(CW // LN):
                sv = srcw_v[b, pl.ds(t * LN, LN)]
                dv = dstw_v[b, pl.ds(t * LN, LN)]
                srci_v[b, pl.ds(2 * t * LN, LN)] = sv & 0xFFFF
                srci_v[b, pl.ds((2 * t + 1) * LN, LN)] = (
                    jnp.right_shift(sv, 16))
                dsti_v[b, pl.ds(2 * t * LN, LN)] = dv & 0xFFFF
                dsti_v[b, pl.ds((2 * t + 1) * LN, LN)] = (
                    jnp.right_shift(dv, 16))

        def start_gather(b):
            pltpu.async_copy(xp_hbm.at[srci_v.at[b]], gbuf.at[b], sem.at[b])

        def finish_scatter(b):
            pltpu.make_async_copy(xp_hbm.at[srci_v.at[b]], gbuf.at[b],
                                  sem.at[b]).wait()
            pltpu.sync_copy(gbuf.at[b], agg_sh.at[dsti_v.at[b]], add=True)

        @pl.when(kw > 0)
        def _prologue():
            load_idx(0, 0)
            start_gather(0)

        def steps(jj, _):
            j0 = 2 * jj
            j1 = 2 * jj + 1

            @pl.when(j1 < kw)
            def _():
                load_idx(j1, 1)
                start_gather(1)
            finish_scatter(0)

            @pl.when(j0 + 2 < kw)
            def _():
                load_idx(j0 + 2, 0)
                start_gather(0)

            @pl.when(j1 < kw)
            def _():
                finish_scatter(1)
            return 0
        lax.fori_loop(0, (kw + 1) // 2, steps, 0)

        plsc.subcore_barrier()

        for off in range(0, rpt, CHUNK):
            sz = min(CHUNK, rpt - off)
            pltpu.sync_copy(agg_sh.at[pl.ds(base + off, sz)],
                            agg_out.at[c, pl.ds(base + off, sz)])

    return sc_aggregate


def _make_sc_count(n_sp, dc, nchunk, rpt):
    mesh = plsc.VectorSubcoreMesh(core_axis_name="c", subcore_axis_name="s",
                                  num_cores=NC, num_subcores=NS)

    @functools.partial(
        pl.kernel,
        out_type=jax.ShapeDtypeStruct((NC, n_sp, dc), jnp.float32),
        mesh=mesh,
        scratch_types=[
            pltpu.VMEM((CW,), jnp.int32),
            pltpu.VMEM((CHUNK,), jnp.int32),
            pltpu.VMEM((CHUNK, dc), jnp.float32),
            pltpu.VMEM((CHUNK, dc), jnp.float32),
            pltpu.VMEM_SHARED((n_sp, dc), jnp.float32),
        ],
    )
    def sc_count(eidx_hbm, cnt_out, dstw_v, dsti_v, ones_v, zb,
                 cnt_sh):
        c = lax.axis_index("c")
        s = lax.axis_index("s")
        w = c * NS + s
        base = s * rpt
        kw = (nchunk - w + NW - 1) // NW

        def fill(i, _):
            def inner(t, __):
                ones_v[i, pl.ds(t * LN, LN)] = jnp.ones((LN,), jnp.float32)
                zb[i, pl.ds(t * LN, LN)] = jnp.zeros((LN,), jnp.float32)
                return 0
            lax.fori_loop(0, dc // LN, inner, 0)
            return 0
        lax.fori_loop(0, CHUNK, fill, 0)

        for off in range(0, rpt, CHUNK):
            sz = min(CHUNK, rpt - off)
            pltpu.sync_copy(zb.at[pl.ds(0, sz)],
                            cnt_sh.at[pl.ds(base + off, sz)])
        plsc.subcore_barrier()

        def step(j, _):
            chunk = j * NW + w
            pltpu.sync_copy(eidx_hbm.at[1, pl.ds(chunk * CW, CW)], dstw_v)
            _unpack_chunk(dstw_v, dsti_v)
            pltpu.sync_copy(ones_v, cnt_sh.at[dsti_v], add=True)
            return 0
        lax.fori_loop(0, kw, step, 0)

        plsc.subcore_barrier()

        for off in range(0, rpt, CHUNK):
            sz = min(CHUNK, rpt - off)
            pltpu.sync_copy(cnt_sh.at[pl.ds(base + off, sz)],
                            cnt_out.at[c, pl.ds(base + off, sz)])

    return sc_count


def kernel(x, edge_index, W_proj, b_proj, W_l, b_l, W_r, W_fc, b_fc):
    n, d = x.shape
    e = edge_index.shape[1]
    c_out = W_fc.shape[1]

    nchunk = e // CHUNK
    assert nchunk * CHUNK == e, "edge count must be a multiple of 128"
    rpt = (-(-n // NS) + 7) // 8 * 8
    n_sp = NS * rpt

    eidx_pk = pl.pallas_call(
        _pack_body,
        out_shape=jax.ShapeDtypeStruct((2, e // 2), jnp.int32),
    )(edge_index)

    xp = pl.pallas_call(
        _proj_body,
        out_shape=jax.ShapeDtypeStruct((n, d), jnp.float32),
    )(x, W_proj, b_proj.reshape(1, d))

    cnt2 = _make_sc_count(n_sp, d, nchunk, rpt)(eidx_pk)
    agg2 = _make_sc_aggregate(n_sp, d, nchunk, rpt)(eidx_pk, xp, cnt2)

    logits = pl.pallas_call(
        functools.partial(_tail_body, n=n),
        out_shape=jax.ShapeDtypeStruct((n, c_out), jnp.float32),
    )(agg2, cnt2, x, W_l, b_l.reshape(1, -1), W_r, W_fc,
      b_fc.reshape(1, -1))
    return logits


# speedup vs baseline: 9.6396x; 1.1625x over previous
import functools

import jax
import jax.numpy as jnp
from jax import lax
from jax.experimental import pallas as pl
from jax.experimental.pallas import tpu as pltpu
from jax.experimental.pallas import tpu_sc as plsc

NC = 2
NS = 16
LN = 16
NW = NC * NS
CHUNK = 128
CW = CHUNK // 2


def _pack_body(e_ref, out_ref):
    half = e_ref.shape[1] // 2
    a = e_ref[:, :half]
    b = e_ref[:, half:]
    out_ref[...] = a | (b << 16)


def _proj_body(x_ref, w_ref, b_ref, out_ref):
    out_ref[...] = jnp.maximum(
        jnp.dot(x_ref[...], w_ref[...], preferred_element_type=jnp.float32)
        + b_ref[...], 0.0)


def _tail_body(agg_ref, cnt_ref, x_ref, wl_ref, bl_ref, wr_ref, wfc_ref,
               bfc_ref, out_ref, *, n):
    agg = agg_ref[0, :n, :] + agg_ref[1, :n, :]
    cntv = cnt_ref[0, :n, :] + cnt_ref[1, :n, :]
    cnt = jnp.sum(cntv, axis=1, keepdims=True) * (1.0 / cntv.shape[1])
    mean = agg / jnp.maximum(cnt, 1.0)
    x = x_ref[...]
    out = (jnp.dot(mean, wl_ref[...], preferred_element_type=jnp.float32)
           + bl_ref[...]
           + jnp.dot(x, wr_ref[...], preferred_element_type=jnp.float32))
    nrm = jnp.sqrt(jnp.sum(out * out, axis=1, keepdims=True))
    out = out / jnp.maximum(nrm, 1e-12)
    h = jnp.maximum(out, 0.0)
    out_ref[...] = (jnp.dot(h, wfc_ref[...], preferred_element_type=jnp.float32)
                    + bfc_ref[...])


def _unpack_chunk(word_v, idx_v, shift_mask=True):
    for t in range(CW // LN):
        v = word_v[pl.ds(t * LN, LN)]
        idx_v[pl.ds(2 * t * LN, LN)] = v & 0xFFFF
        idx_v[pl.ds((2 * t + 1) * LN, LN)] = jnp.right_shift(v, 16)


def _make_sc_aggregate(n_sp, d, nchunk, rpt):
    mesh = plsc.VectorSubcoreMesh(core_axis_name="c", subcore_axis_name="s",
                                  num_cores=NC, num_subcores=NS)

    @functools.partial(
        pl.kernel,
        out_type=jax.ShapeDtypeStruct((NC, n_sp, d), jnp.float32),
        mesh=mesh,
        scratch_types=[
            pltpu.VMEM((2, CW), jnp.int32),
            pltpu.VMEM((2, CW), jnp.int32),
            pltpu.VMEM((2, CHUNK), jnp.int32),
            pltpu.VMEM((2, CHUNK), jnp.int32),
            pltpu.VMEM((2, CHUNK, d), jnp.float32),
            pltpu.VMEM_SHARED((n_sp, d), jnp.float32),
            pltpu.SemaphoreType.DMA((2,)),
            pltpu.SemaphoreType.DMA((2,)),
        ],
    )
    def sc_aggregate(eidx_hbm, xp_hbm, dep_hbm, agg_out,
                     srcw_v, dstw_v, srci_v, dsti_v, gbuf, agg_sh, sem,
                     sem_i):
        del dep_hbm
        c = lax.axis_index("c")
        s = lax.axis_index("s")
        w = c * NS + s
        base = s * rpt
        kw = (nchunk - w + NW - 1) // NW

        def fill(i, _):
            def inner(t, __):
                gbuf[0, i, pl.ds(t * LN, LN)] = jnp.zeros((LN,), jnp.float32)
                return 0
            lax.fori_loop(0, d // LN, inner, 0)
            return 0
        lax.fori_loop(0, CHUNK, fill, 0)

        for off in range(0, rpt, CHUNK):
            sz = min(CHUNK, rpt - off)
            pltpu.sync_copy(gbuf.at[0, pl.ds(0, sz)],
                            agg_sh.at[pl.ds(base + off, sz)])
        plsc.subcore_barrier()

        def start_idx(j, b):
            chunk = j * NW + w
            pltpu.async_copy(eidx_hbm.at[0, pl.ds(chunk * CW, CW)],
                             srcw_v.at[b], sem_i.at[b])
            pltpu.async_copy(eidx_hbm.at[1, pl.ds(chunk * CW, CW)],
                             dstw_v.at[b], sem_i.at[b])

        def wait_unpack_idx(j, b):
            chunk = j * NW + w
            pltpu.make_async_copy(eidx_hbm.at[0, pl.ds(chunk * CW, CW)],
                                  srcw_v.at[b], sem_i.at[b]).wait()
            pltpu.make_async_copy(eidx_hbm.at[1, pl.ds(chunk * CW, CW)],
                                  dstw_v.at[b], sem_i.at[b]).wait()
            for t in range(CW // LN):
                sv = srcw_v[b, pl.ds(t * LN, LN)]
                dv = dstw_v[b, pl.ds(t * LN, LN)]
                srci_v[b, pl.ds(2 * t * LN, LN)] = sv & 0xFFFF
                srci_v[b, pl.ds((2 * t + 1) * LN, LN)] = (
                    jnp.right_shift(sv, 16))
                dsti_v[b, pl.ds(2 * t * LN, LN)] = dv & 0xFFFF
                dsti_v[b, pl.ds((2 * t + 1) * LN, LN)] = (
                    jnp.right_shift(dv, 16))

        def start_gather(b):
            pltpu.async_copy(xp_hbm.at[srci_v.at[b]], gbuf.at[b], sem.at[b])

        def finish_scatter(b):
            pltpu.make_async_copy(xp_hbm.at[srci_v.at[b]], gbuf.at[b],
                                  sem.at[b]).wait()
            pltpu.sync_copy(gbuf.at[b], agg_sh.at[dsti_v.at[b]], add=True)

        @pl.when(kw > 0)
        def _prologue():
            start_idx(0, 0)
            wait_unpack_idx(0, 0)
            start_gather(0)

        @pl.when(kw > 1)
        def _prologue2():
            start_idx(1, 1)

        def steps(jj, _):
            j0 = 2 * jj
            j1 = 2 * jj + 1

            @pl.when(j1 < kw)
            def _():
                wait_unpack_idx(j1, 1)
                start_gather(1)

            @pl.when(j0 + 2 < kw)
            def _():
                start_idx(j0 + 2, 0)
            finish_scatter(0)

            @pl.when(j0 + 2 < kw)
            def _():
                wait_unpack_idx(j0 + 2, 0)
                start_gather(0)

            @pl.when(j1 + 2 < kw)
            def _():
                start_idx(j1 + 2, 1)

            @pl.when(j1 < kw)
            def _():
                finish_scatter(1)
            return 0
        lax.fori_loop(0, (kw + 1) // 2, steps, 0)

        plsc.subcore_barrier()

        for off in range(0, rpt, CHUNK):
            sz = min(CHUNK, rpt - off)
            pltpu.sync_copy(agg_sh.at[pl.ds(base + off, sz)],
                            agg_out.at[c, pl.ds(base + off, sz)])

    return sc_aggregate


def _make_sc_count(n_sp, dc, nchunk, rpt):
    mesh = plsc.VectorSubcoreMesh(core_axis_name="c", subcore_axis_name="s",
                                  num_cores=NC, num_subcores=NS)

    @functools.partial(
        pl.kernel,
        out_type=jax.ShapeDtypeStruct((NC, n_sp, dc), jnp.float32),
        mesh=mesh,
        scratch_types=[
            pltpu.VMEM((CW,), jnp.int32),
            pltpu.VMEM((CHUNK,), jnp.int32),
            pltpu.VMEM((CHUNK, dc), jnp.float32),
            pltpu.VMEM((CHUNK, dc), jnp.float32),
            pltpu.VMEM_SHARED((n_sp, dc), jnp.float32),
        ],
    )
    def sc_count(eidx_hbm, cnt_out, dstw_v, dsti_v, ones_v, zb,
                 cnt_sh):
        c = lax.axis_index("c")
        s = lax.axis_index("s")
        w = c * NS + s
        base = s * rpt
        kw = (nchunk - w + NW - 1) // NW

        def fill(i, _):
            def inner(t, __):
                ones_v[i, pl.ds(t * LN, LN)] = jnp.ones((LN,), jnp.float32)
                zb[i, pl.ds(t * LN, LN)] = jnp.zeros((LN,), jnp.float32)
                return 0
            lax.fori_loop(0, dc // LN, inner, 0)
            return 0
        lax.fori_loop(0, CHUNK, fill, 0)

        for off in range(0, rpt, CHUNK):
            sz = min(CHUNK, rpt - off)
            pltpu.sync_copy(zb.at[pl.ds(0, sz)],
                            cnt_sh.at[pl.ds(base + off, sz)])
        plsc.subcore_barrier()

        def step(j, _):
            chunk = j * NW + w
            pltpu.sync_copy(eidx_hbm.at[1, pl.ds(chunk * CW, CW)], dstw_v)
            _unpack_chunk(dstw_v, dsti_v)
            pltpu.sync_copy(ones_v, cnt_sh.at[dsti_v], add=True)
            return 0
        lax.fori_loop(0, kw, step, 0)

        plsc.subcore_barrier()

        for off in range(0, rpt, CHUNK):
            sz = min(CHUNK, rpt - off)
            pltpu.sync_copy(cnt_sh.at[pl.ds(base + off, sz)],
                            cnt_out.at[c, pl.ds(base + off, sz)])

    return sc_count


def kernel(x, edge_index, W_proj, b_proj, W_l, b_l, W_r, W_fc, b_fc):
    n, d = x.shape
    e = edge_index.shape[1]
    c_out = W_fc.shape[1]

    nchunk = e // CHUNK
    assert nchunk * CHUNK == e, "edge count must be a multiple of 128"
    rpt = (-(-n // NS) + 7) // 8 * 8
    n_sp = NS * rpt

    eidx_pk = pl.pallas_call(
        _pack_body,
        out_shape=jax.ShapeDtypeStruct((2, e // 2), jnp.int32),
    )(edge_index)

    xp = pl.pallas_call(
        _proj_body,
        out_shape=jax.ShapeDtypeStruct((n, d), jnp.float32),
    )(x, W_proj, b_proj.reshape(1, d))

    cnt2 = _make_sc_count(n_sp, d, nchunk, rpt)(eidx_pk)
    agg2 = _make_sc_aggregate(n_sp, d, nchunk, rpt)(eidx_pk, xp, cnt2)

    logits = pl.pallas_call(
        functools.partial(_tail_body, n=n),
        out_shape=jax.ShapeDtypeStruct((n, c_out), jnp.float32),
    )(agg2, cnt2, x, W_l, b_l.reshape(1, -1), W_r, W_fc,
      b_fc.reshape(1, -1))
    return logits


# speedup vs baseline: 11.0058x; 1.1417x over previous
import functools

import jax
import jax.numpy as jnp
from jax import lax
from jax.experimental import pallas as pl
from jax.experimental.pallas import tpu as pltpu
from jax.experimental.pallas import tpu_sc as plsc

NC = 2
NS = 16
LN = 16
NW = NC * NS
CHUNK = 128
CW = CHUNK // 2


def _pack_body(e_ref, out_ref):
    half = e_ref.shape[1] // 2
    a = e_ref[:, :half]
    b = e_ref[:, half:]
    out_ref[...] = a | (b << 16)


def _proj_body(x_ref, w_ref, b_ref, out_ref):
    out_ref[...] = jnp.maximum(
        jnp.dot(x_ref[...], w_ref[...], preferred_element_type=jnp.float32)
        + b_ref[...], 0.0)


def _tail_body(agg_ref, cnt_ref, x_ref, wl_ref, bl_ref, wr_ref, wfc_ref,
               bfc_ref, out_ref, *, n):
    agg = agg_ref[0, :n, :] + agg_ref[1, :n, :]
    cntv = cnt_ref[0, :n, :] + cnt_ref[1, :n, :]
    cnt = jnp.sum(cntv, axis=1, keepdims=True) * (1.0 / cntv.shape[1])
    mean = agg / jnp.maximum(cnt, 1.0)
    x = x_ref[...]
    out = (jnp.dot(mean, wl_ref[...], preferred_element_type=jnp.float32)
           + bl_ref[...]
           + jnp.dot(x, wr_ref[...], preferred_element_type=jnp.float32))
    nrm = jnp.sqrt(jnp.sum(out * out, axis=1, keepdims=True))
    out = out / jnp.maximum(nrm, 1e-12)
    h = jnp.maximum(out, 0.0)
    out_ref[...] = (jnp.dot(h, wfc_ref[...], preferred_element_type=jnp.float32)
                    + bfc_ref[...])


def _unpack_chunk(word_v, idx_v, shift_mask=True):
    for t in range(CW // LN):
        v = word_v[pl.ds(t * LN, LN)]
        idx_v[pl.ds(2 * t * LN, LN)] = v & 0xFFFF
        idx_v[pl.ds((2 * t + 1) * LN, LN)] = jnp.right_shift(v, 16)


def _make_sc_aggregate(n_sp, d, nchunk, rpt):
    mesh = plsc.VectorSubcoreMesh(core_axis_name="c", subcore_axis_name="s",
                                  num_cores=NC, num_subcores=NS)

    @functools.partial(
        pl.kernel,
        out_type=jax.ShapeDtypeStruct((NC, n_sp, d), jnp.float32),
        mesh=mesh,
        scratch_types=[
            pltpu.VMEM((2, CW), jnp.int32),
            pltpu.VMEM((2, CW), jnp.int32),
            pltpu.VMEM((2, CHUNK), jnp.int32),
            pltpu.VMEM((2, CHUNK), jnp.int32),
            pltpu.VMEM((2, CHUNK, d), jnp.float32),
            pltpu.VMEM_SHARED((n_sp, d), jnp.float32),
            pltpu.SemaphoreType.DMA((2,)),
            pltpu.SemaphoreType.DMA((2,)),
        ],
    )
    def sc_aggregate(eidx_hbm, xp_hbm, dep_hbm, agg_out,
                     srcw_v, dstw_v, srci_v, dsti_v, gbuf, agg_sh, sem,
                     sem_i):
        del dep_hbm
        c = lax.axis_index("c")
        s = lax.axis_index("s")
        w = c * NS + s
        base = s * rpt
        kw = (nchunk - w + NW - 1) // NW

        def fill(i, _):
            def inner(t, __):
                gbuf[0, i, pl.ds(t * LN, LN)] = jnp.zeros((LN,), jnp.float32)
                return 0
            lax.fori_loop(0, d // LN, inner, 0)
            return 0
        lax.fori_loop(0, CHUNK, fill, 0)

        for off in range(0, rpt, CHUNK):
            sz = min(CHUNK, rpt - off)
            pltpu.sync_copy(gbuf.at[0, pl.ds(0, sz)],
                            agg_sh.at[pl.ds(base + off, sz)])
        plsc.subcore_barrier()

        def start_idx(j, b):
            chunk = j * NW + w
            pltpu.async_copy(eidx_hbm.at[0, pl.ds(chunk * CW, CW)],
                             srcw_v.at[b], sem_i.at[b])
            pltpu.async_copy(eidx_hbm.at[1, pl.ds(chunk * CW, CW)],
                             dstw_v.at[b], sem_i.at[b])

        def wait_unpack_idx(j, b):
            chunk = j * NW + w
            pltpu.make_async_copy(eidx_hbm.at[0, pl.ds(chunk * CW, CW)],
                                  srcw_v.at[b], sem_i.at[b]).wait()
            pltpu.make_async_copy(eidx_hbm.at[1, pl.ds(chunk * CW, CW)],
                                  dstw_v.at[b], sem_i.at[b]).wait()
            for t in range(CW // LN):
                sv = srcw_v[b, pl.ds(t * LN, LN)]
                dv = dstw_v[b, pl.ds(t * LN, LN)]
                srci_v[b, pl.ds(2 * t * LN, LN)] = sv & 0xFFFF
                srci_v[b, pl.ds((2 * t + 1) * LN, LN)] = (
                    jnp.right_shift(sv, 16))
                dsti_v[b, pl.ds(2 * t * LN, LN)] = dv & 0xFFFF
                dsti_v[b, pl.ds((2 * t + 1) * LN, LN)] = (
                    jnp.right_shift(dv, 16))

        def start_gather(b):
            pltpu.async_copy(xp_hbm.at[srci_v.at[b]], gbuf.at[b], sem.at[b])

        def finish_scatter(b):
            pltpu.make_async_copy(xp_hbm.at[srci_v.at[b]], gbuf.at[b],
                                  sem.at[b]).wait()
            pltpu.sync_copy(gbuf.at[b], agg_sh.at[dsti_v.at[b]], add=True)

        @pl.when(kw > 0)
        def _prologue():
            start_idx(0, 0)
            wait_unpack_idx(0, 0)
            start_gather(0)

        @pl.when(kw > 1)
        def _prologue2():
            start_idx(1, 1)

        def steps(jj, _):
            j0 = 2 * jj
            j1 = 2 * jj + 1

            @pl.when(j1 < kw)
            def _():
                wait_unpack_idx(j1, 1)
                start_gather(1)

            @pl.when(j0 + 2 < kw)
            def _():
                start_idx(j0 + 2, 0)
            finish_scatter(0)

            @pl.when(j0 + 2 < kw)
            def _():
                wait_unpack_idx(j0 + 2, 0)
                start_gather(0)

            @pl.when(j1 + 2 < kw)
            def _():
                start_idx(j1 + 2, 1)

            @pl.when(j1 < kw)
            def _():
                finish_scatter(1)
            return 0
        lax.fori_loop(0, (kw + 1) // 2, steps, 0)

        plsc.subcore_barrier()

        for off in range(0, rpt, CHUNK):
            sz = min(CHUNK, rpt - off)
            pltpu.sync_copy(agg_sh.at[pl.ds(base + off, sz)],
                            agg_out.at[c, pl.ds(base + off, sz)])

    return sc_aggregate


def _make_sc_count(n_sp, dc, nchunk, rpt):
    mesh = plsc.VectorSubcoreMesh(core_axis_name="c", subcore_axis_name="s",
                                  num_cores=NC, num_subcores=NS)

    @functools.partial(
        pl.kernel,
        out_type=jax.ShapeDtypeStruct((NC, n_sp, dc), jnp.float32),
        mesh=mesh,
        scratch_types=[
            pltpu.VMEM((2, CW), jnp.int32),
            pltpu.VMEM((2, CHUNK), jnp.int32),
            pltpu.VMEM((CHUNK, dc), jnp.float32),
            pltpu.VMEM((CHUNK, dc), jnp.float32),
            pltpu.VMEM_SHARED((n_sp, dc), jnp.float32),
            pltpu.SemaphoreType.DMA((2,)),
        ],
    )
    def sc_count(eidx_hbm, cnt_out, dstw_v, dsti_v, ones_v, zb,
                 cnt_sh, sem_i):
        c = lax.axis_index("c")
        s = lax.axis_index("s")
        w = c * NS + s
        base = s * rpt
        kw = (nchunk - w + NW - 1) // NW

        def fill(i, _):
            def inner(t, __):
                ones_v[i, pl.ds(t * LN, LN)] = jnp.ones((LN,), jnp.float32)
                zb[i, pl.ds(t * LN, LN)] = jnp.zeros((LN,), jnp.float32)
                return 0
            lax.fori_loop(0, dc // LN, inner, 0)
            return 0
        lax.fori_loop(0, CHUNK, fill, 0)

        for off in range(0, rpt, CHUNK):
            sz = min(CHUNK, rpt - off)
            pltpu.sync_copy(zb.at[pl.ds(0, sz)],
                            cnt_sh.at[pl.ds(base + off, sz)])
        plsc.subcore_barrier()

        def start_idx(j, b):
            chunk = j * NW + w
            pltpu.async_copy(eidx_hbm.at[1, pl.ds(chunk * CW, CW)],
                             dstw_v.at[b], sem_i.at[b])

        def wait_unpack_idx(j, b):
            chunk = j * NW + w
            pltpu.make_async_copy(eidx_hbm.at[1, pl.ds(chunk * CW, CW)],
                                  dstw_v.at[b], sem_i.at[b]).wait()
            for t in range(CW // LN):
                dv = dstw_v[b, pl.ds(t * LN, LN)]
                dsti_v[b, pl.ds(2 * t * LN, LN)] = dv & 0xFFFF
                dsti_v[b, pl.ds((2 * t + 1) * LN, LN)] = (
                    jnp.right_shift(dv, 16))

        def scatter_ones(b):
            pltpu.sync_copy(ones_v, cnt_sh.at[dsti_v.at[b]], add=True)

        @pl.when(kw > 0)
        def _prologue():
            start_idx(0, 0)

        def steps(jj, _):
            j0 = 2 * jj
            j1 = 2 * jj + 1
            wait_unpack_idx(j0, 0)

            @pl.when(j1 < kw)
            def _():
                start_idx(j1, 1)
            scatter_ones(0)

            @pl.when(j1 < kw)
            def _():
                wait_unpack_idx(j1, 1)

                @pl.when(j0 + 2 < kw)
                def _():
                    start_idx(j0 + 2, 0)
                scatter_ones(1)
            return 0
        lax.fori_loop(0, (kw + 1) // 2, steps, 0)

        plsc.subcore_barrier()

        for off in range(0, rpt, CHUNK):
            sz = min(CHUNK, rpt - off)
            pltpu.sync_copy(cnt_sh.at[pl.ds(base + off, sz)],
                            cnt_out.at[c, pl.ds(base + off, sz)])

    return sc_count


def kernel(x, edge_index, W_proj, b_proj, W_l, b_l, W_r, W_fc, b_fc):
    n, d = x.shape
    e = edge_index.shape[1]
    c_out = W_fc.shape[1]

    nchunk = e // CHUNK
    assert nchunk * CHUNK == e, "edge count must be a multiple of 128"
    rpt = (-(-n // NS) + 7) // 8 * 8
    n_sp = NS * rpt

    eidx_pk = pl.pallas_call(
        _pack_body,
        out_shape=jax.ShapeDtypeStruct((2, e // 2), jnp.int32),
    )(edge_index)

    xp = pl.pallas_call(
        _proj_body,
        out_shape=jax.ShapeDtypeStruct((n, d), jnp.float32),
    )(x, W_proj, b_proj.reshape(1, d))

    cnt2 = _make_sc_count(n_sp, d, nchunk, rpt)(eidx_pk)
    agg2 = _make_sc_aggregate(n_sp, d, nchunk, rpt)(eidx_pk, xp, cnt2)

    logits = pl.pallas_call(
        functools.partial(_tail_body, n=n),
        out_shape=jax.ShapeDtypeStruct((n, c_out), jnp.float32),
    )(agg2, cnt2, x, W_l, b_l.reshape(1, -1), W_r, W_fc,
      b_fc.reshape(1, -1))
    return logits
